# baseline (speedup 1.0000x reference)
import functools

import jax
import jax.numpy as jnp
from jax import lax
from jax.experimental import pallas as pl
from jax.experimental.pallas import tpu as pltpu
from jax.experimental.pallas import tpu_sc as plsc

MAXLEN = 200
VOCAB = 100000
D = 128
BATCH = 1024

NC = 2
NS = 16
NW = NC * NS

N = BATCH * MAXLEN
RW = N // NW
CHUNK = 256
NCHUNK = RW // CHUNK
LANES = 16
SUBV = D // LANES


NBUF = 3


def _body(ids_hbm, tok_hbm, pos_hbm, out_hbm, idx_v, pos_v, *scr):
    wid = lax.axis_index("s") * NC + lax.axis_index("c")
    base = wid * RW
    bufs = scr[:NBUF]
    gsems = scr[NBUF:2 * NBUF]
    ssems = scr[2 * NBUF:3 * NBUF]
    psems = scr[3 * NBUF:4 * NBUF]

    pltpu.sync_copy(ids_hbm.at[wid], idx_v)

    @pl.when(lax.axis_index("s") == 0)
    def _():
        for off in range(0, MAXLEN + CHUNK, MAXLEN):
            ln = min(MAXLEN, MAXLEN + CHUNK - off)
            pltpu.sync_copy(
                pos_hbm.at[pl.ds(0, ln)], pos_v.at[pl.ds(off, ln)]
            )

    plsc.subcore_barrier()

    def prefill_and_gather_add(c, b):
        p0 = lax.rem(c * CHUNK, MAXLEN)
        pltpu.async_copy(pos_v.at[pl.ds(p0, CHUNK)], bufs[b], psems[b])
        pltpu.make_async_copy(
            pos_v.at[pl.ds(0, CHUNK)], bufs[b], psems[b]
        ).wait()
        pltpu.async_copy(
            tok_hbm.at[idx_v.at[0, pl.ds(c * CHUNK, CHUNK)]],
            bufs[b], gsems[b], add=True,
        )

    for b in range(NBUF - 1):
        prefill_and_gather_add(b, b)

    ngroups = -(-NCHUNK // NBUF)

    @pl.loop(0, ngroups * NBUF, step=NBUF)
    def _group(g):
        for b in range(NBUF):
            c = g + b
            nb = (b + NBUF - 1) % NBUF

            @pl.when((c + NBUF - 1 < NCHUNK) & (c >= 1))
            def _():
                pltpu.make_async_copy(
                    bufs[nb], out_hbm.at[pl.ds(0, CHUNK)], ssems[nb]
                ).wait()

            @pl.when(c + NBUF - 1 < NCHUNK)
            def _():
                prefill_and_gather_add(c + NBUF - 1, nb)

            @pl.when(c < NCHUNK)
            def _():
                buf = bufs[b]
                pltpu.make_async_copy(
                    tok_hbm.at[idx_v.at[0, pl.ds(c * CHUNK, CHUNK)]],
                    buf, gsems[b],
                ).wait()
                pltpu.async_copy(
                    buf, out_hbm.at[pl.ds(base + c * CHUNK, CHUNK)], ssems[b]
                )

    for b in range(NBUF):
        pltpu.make_async_copy(
            bufs[b], out_hbm.at[pl.ds(0, CHUNK)], ssems[b]
        ).wait()


def kernel(input_ids, token_table, pos_table):
    ids = input_ids.astype(jnp.int32).reshape(NW, 1, RW)
    mesh = plsc.VectorSubcoreMesh(
        core_axis_name="c", subcore_axis_name="s", num_cores=NC, num_subcores=NS
    )
    run = pl.kernel(
        _body,
        out_type=jax.ShapeDtypeStruct((N, D), jnp.float32),
        mesh=mesh,
        scratch_types=[
            pltpu.VMEM((1, RW), jnp.int32),
            pltpu.VMEM_SHARED((MAXLEN + CHUNK, D), jnp.float32),
        ] + [pltpu.VMEM((CHUNK, D), jnp.float32)] * NBUF
          + [pltpu.SemaphoreType.DMA] * (3 * NBUF),
    )
    out = run(ids, token_table, pos_table)
    return out.reshape(BATCH, MAXLEN, D)


# speedup vs baseline: 1.0200x; 1.0200x over previous
import functools

import jax
import jax.numpy as jnp
from jax import lax
from jax.experimental import pallas as pl
from jax.experimental.pallas import tpu as pltpu
from jax.experimental.pallas import tpu_sc as plsc

MAXLEN = 200
VOCAB = 100000
D = 128
BATCH = 1024

NC = 2
NS = 16
NW = NC * NS

N = BATCH * MAXLEN
RW = N // NW
CHUNK = 256
NCHUNK = RW // CHUNK
LANES = 16
SUBV = D // LANES


NBUF = 3


def _body(ids_hbm, tok_hbm, pos_hbm, out_hbm, idx_v, pos_v, *scr):
    wid = lax.axis_index("s") * NC + lax.axis_index("c")
    base = wid * RW
    bufs = scr[:NBUF]
    gsems = scr[NBUF:2 * NBUF]
    ssems = scr[2 * NBUF:3 * NBUF]
    psems = scr[3 * NBUF:4 * NBUF]

    segs = []
    for off in range(0, MAXLEN + CHUNK, MAXLEN):
        segs.append((off, min(MAXLEN, MAXLEN + CHUNK - off)))

    @pl.when(lax.axis_index("s") == 0)
    def _():
        for off, ln in segs:
            pltpu.async_copy(
                pos_hbm.at[pl.ds(0, ln)], pos_v.at[pl.ds(off, ln)], psems[0]
            )

    pltpu.sync_copy(ids_hbm.at[wid], idx_v)

    @pl.when(lax.axis_index("s") == 0)
    def _():
        for off, ln in segs:
            pltpu.make_async_copy(
                pos_hbm.at[pl.ds(0, ln)], pos_v.at[pl.ds(off, ln)], psems[0]
            ).wait()

    plsc.subcore_barrier()

    def prefill_and_gather_add(c, b):
        p0 = lax.rem(c * CHUNK, MAXLEN)
        pltpu.async_copy(pos_v.at[pl.ds(p0, CHUNK)], bufs[b], psems[b])
        pltpu.make_async_copy(
            pos_v.at[pl.ds(0, CHUNK)], bufs[b], psems[b]
        ).wait()
        pltpu.async_copy(
            tok_hbm.at[idx_v.at[0, pl.ds(c * CHUNK, CHUNK)]],
            bufs[b], gsems[b], add=True,
        )

    for b in range(NBUF - 1):
        prefill_and_gather_add(b, b)

    ngroups = -(-NCHUNK // NBUF)

    @pl.loop(0, ngroups * NBUF, step=NBUF)
    def _group(g):
        for b in range(NBUF):
            c = g + b
            nb = (b + NBUF - 1) % NBUF

            @pl.when((c + NBUF - 1 < NCHUNK) & (c >= 1))
            def _():
                pltpu.make_async_copy(
                    bufs[nb], out_hbm.at[pl.ds(0, CHUNK)], ssems[nb]
                ).wait()

            @pl.when(c + NBUF - 1 < NCHUNK)
            def _():
                prefill_and_gather_add(c + NBUF - 1, nb)

            @pl.when(c < NCHUNK)
            def _():
                buf = bufs[b]
                pltpu.make_async_copy(
                    tok_hbm.at[idx_v.at[0, pl.ds(c * CHUNK, CHUNK)]],
                    buf, gsems[b],
                ).wait()
                pltpu.async_copy(
                    buf, out_hbm.at[pl.ds(base + c * CHUNK, CHUNK)], ssems[b]
                )

    for b in range(NBUF):
        pltpu.make_async_copy(
            bufs[b], out_hbm.at[pl.ds(0, CHUNK)], ssems[b]
        ).wait()


def kernel(input_ids, token_table, pos_table):
    ids = input_ids.astype(jnp.int32).reshape(NW, 1, RW)
    mesh = plsc.VectorSubcoreMesh(
        core_axis_name="c", subcore_axis_name="s", num_cores=NC, num_subcores=NS
    )
    run = pl.kernel(
        _body,
        out_type=jax.ShapeDtypeStruct((N, D), jnp.float32),
        mesh=mesh,
        scratch_types=[
            pltpu.VMEM((1, RW), jnp.int32),
            pltpu.VMEM_SHARED((MAXLEN + CHUNK, D), jnp.float32),
        ] + [pltpu.VMEM((CHUNK, D), jnp.float32)] * NBUF
          + [pltpu.SemaphoreType.DMA] * (3 * NBUF),
    )
    out = run(ids, token_table, pos_table)
    return out.reshape(BATCH, MAXLEN, D)
